# baseline (speedup 1.0000x reference)
import jax
import jax.numpy as jnp
from jax.experimental import pallas as pl
from jax.experimental.pallas import tpu as pltpu

_M = 65536
_K = 100
_N = 100
_CHUNK = 2048
_NCHUNKS = _M // _CHUNK
_NBUF_IN = 4
_NBUF_OUT = 6


def _pipelined_kernel(xt_hbm, w_vmem, b_vmem, o_hbm,
                      xbuf, obuf, in_sems, out_sems):
    def in_copy(i):
        slot = i % _NBUF_IN
        return pltpu.make_async_copy(
            xt_hbm.at[:, pl.ds(i * _CHUNK, _CHUNK)],
            xbuf.at[slot],
            in_sems.at[slot],
        )

    def out_copy(i):
        slot = i % _NBUF_OUT
        return pltpu.make_async_copy(
            obuf.at[slot],
            o_hbm.at[:, pl.ds(i * _CHUNK, _CHUNK)],
            out_sems.at[slot],
        )

    for i in range(_NBUF_IN):
        in_copy(i).start()

    for i in range(_NCHUNKS):
        in_copy(i).wait()
        if i >= _NBUF_OUT:
            out_copy(i - _NBUF_OUT).wait()
        acc = jnp.dot(w_vmem[...], xbuf[i % _NBUF_IN],
                      preferred_element_type=jnp.float32)
        obuf[i % _NBUF_OUT] = jnp.maximum(acc + b_vmem[...], 0.0)
        out_copy(i).start()
        if i + _NBUF_IN < _NCHUNKS:
            in_copy(i + _NBUF_IN).start()

    for i in range(max(_NCHUNKS - _NBUF_OUT, 0), _NCHUNKS):
        out_copy(i).wait()


def kernel(x, W1, b1):
    xt = x.T
    b = b1[:, None]
    ht = pl.pallas_call(
        _pipelined_kernel,
        in_specs=[
            pl.BlockSpec(memory_space=pl.ANY),
            pl.BlockSpec(memory_space=pltpu.MemorySpace.VMEM),
            pl.BlockSpec(memory_space=pltpu.MemorySpace.VMEM),
        ],
        out_specs=pl.BlockSpec(memory_space=pl.ANY),
        out_shape=jax.ShapeDtypeStruct((_N, _M), jnp.float32),
        scratch_shapes=[
            pltpu.VMEM((_NBUF_IN, _K, _CHUNK), jnp.float32),
            pltpu.VMEM((_NBUF_OUT, _N, _CHUNK), jnp.float32),
            pltpu.SemaphoreType.DMA((_NBUF_IN,)),
            pltpu.SemaphoreType.DMA((_NBUF_OUT,)),
        ],
    )(xt, W1, b)
    h = ht.T
    zero = jnp.zeros((), dtype=jnp.float32)
    return (h, zero, zero, zero)


# speedup vs baseline: 1.0867x; 1.0867x over previous
import jax
import jax.numpy as jnp
from jax.experimental import pallas as pl
from jax.experimental.pallas import tpu as pltpu

_M = 65536
_K = 100
_N = 100
_CHUNK = 8192
_NCHUNKS = _M // _CHUNK
_NBUF_IN = 4
_NBUF_OUT = 4


def _pipelined_kernel(xt_hbm, w_vmem, b_vmem, o_hbm,
                      xbuf, obuf, in_sems, out_sems):
    def in_copy(i):
        slot = i % _NBUF_IN
        return pltpu.make_async_copy(
            xt_hbm.at[:, pl.ds(i * _CHUNK, _CHUNK)],
            xbuf.at[slot],
            in_sems.at[slot],
        )

    def out_copy(i):
        slot = i % _NBUF_OUT
        return pltpu.make_async_copy(
            obuf.at[slot],
            o_hbm.at[:, pl.ds(i * _CHUNK, _CHUNK)],
            out_sems.at[slot],
        )

    for i in range(_NBUF_IN):
        in_copy(i).start()

    for i in range(_NCHUNKS):
        in_copy(i).wait()
        if i >= _NBUF_OUT:
            out_copy(i - _NBUF_OUT).wait()
        acc = jnp.dot(w_vmem[...], xbuf[i % _NBUF_IN],
                      preferred_element_type=jnp.float32)
        obuf[i % _NBUF_OUT] = jnp.maximum(acc + b_vmem[...], 0.0)
        out_copy(i).start()
        if i + _NBUF_IN < _NCHUNKS:
            in_copy(i + _NBUF_IN).start()

    for i in range(max(_NCHUNKS - _NBUF_OUT, 0), _NCHUNKS):
        out_copy(i).wait()


def kernel(x, W1, b1):
    xt = x.T
    b = b1[:, None]
    ht = pl.pallas_call(
        _pipelined_kernel,
        in_specs=[
            pl.BlockSpec(memory_space=pl.ANY),
            pl.BlockSpec(memory_space=pltpu.MemorySpace.VMEM),
            pl.BlockSpec(memory_space=pltpu.MemorySpace.VMEM),
        ],
        out_specs=pl.BlockSpec(memory_space=pl.ANY),
        out_shape=jax.ShapeDtypeStruct((_N, _M), jnp.float32),
        scratch_shapes=[
            pltpu.VMEM((_NBUF_IN, _K, _CHUNK), jnp.float32),
            pltpu.VMEM((_NBUF_OUT, _N, _CHUNK), jnp.float32),
            pltpu.SemaphoreType.DMA((_NBUF_IN,)),
            pltpu.SemaphoreType.DMA((_NBUF_OUT,)),
        ],
    )(xt, W1, b)
    h = ht.T
    zero = jnp.zeros((), dtype=jnp.float32)
    return (h, zero, zero, zero)
